# baseline (speedup 1.0000x reference)
import functools

import jax
import jax.numpy as jnp
from jax import lax
from jax.experimental import pallas as pl
from jax.experimental.pallas import tpu as pltpu
from jax.experimental.pallas import tpu_sc as plsc

EMBED = 256
HIDDEN = 128
LQ = 20
LT = 50
LTP = 56
EPS = 1e-08

NC = 2
NS = 16
NW = NC * NS

GQ = 4
GT = 2
NV = EMBED // 16


def _accumulate(rows_v, pool_v, base_row, elem, L):
    def body(r, accs):
        row = base_row + r
        return tuple(accs[v] + rows_v[row, pl.ds(v * 16, 16)]
                     for v in range(NV))
    init = tuple(rows_v[base_row, pl.ds(v * 16, 16)] for v in range(NV))
    accs = lax.fori_loop(1, L, body, init)
    for v in range(NV):
        pool_v[elem, pl.ds(v * 16, 16)] = accs[v]


def _pool_field(emb, idx_hbm, out_hbm, wid, ew, L, Lp, g, nchunks):
    R = g * Lp

    def scoped(idx_v, rows_v, pool_v, gsem, osem):
        pltpu.sync_copy(idx_hbm.at[wid], idx_v)
        for b in range(2):
            pltpu.async_copy(emb.at[idx_v.at[b]], rows_v.at[b], gsem.at[b])

        def chunk_body(cc, carry):
            for b in range(2):
                c = cc * 2 + b
                pltpu.make_async_copy(
                    emb.at[idx_v.at[c]], rows_v.at[b], gsem.at[b]).wait()
                @pl.when(c >= 2)
                def _():
                    pltpu.make_async_copy(
                        pool_v.at[b],
                        out_hbm.at[pl.ds(wid * ew + (c - 2) * g, g)],
                        osem.at[b]).wait()
                for e in range(g):
                    _accumulate(rows_v.at[b], pool_v.at[b], e * Lp, e, L)
                pltpu.async_copy(
                    pool_v.at[b],
                    out_hbm.at[pl.ds(wid * ew + c * g, g)],
                    osem.at[b])
                @pl.when(c + 2 < nchunks)
                def _():
                    pltpu.async_copy(
                        emb.at[idx_v.at[c + 2]], rows_v.at[b], gsem.at[b])
            return carry

        lax.fori_loop(0, nchunks // 2, chunk_body, 0)
        for b in range(2):
            c = nchunks - 2 + b
            pltpu.make_async_copy(
                pool_v.at[b],
                out_hbm.at[pl.ds(wid * ew + c * g, g)],
                osem.at[b]).wait()

    pl.run_scoped(
        scoped,
        pltpu.VMEM((nchunks, R), jnp.int32),
        pltpu.VMEM((2, R, EMBED), jnp.float32),
        pltpu.VMEM((2, g, EMBED), jnp.float32),
        pltpu.SemaphoreType.DMA((2,)),
        pltpu.SemaphoreType.DMA((2,)),
    )


@functools.lru_cache(maxsize=None)
def _build_sc_pool(batch, interpret=False):
    ew = batch // NW
    ncq = ew // GQ
    nct = ew // GT

    def body(emb, qidx, pidx, nidx, qout, pout, nout):
        wid = lax.axis_index("c") * NS + lax.axis_index("s")
        _pool_field(emb, qidx, qout, wid, ew, LQ, LQ, GQ, ncq)
        _pool_field(emb, pidx, pout, wid, ew, LT, LTP, GT, nct)
        _pool_field(emb, nidx, nout, wid, ew, LT, LTP, GT, nct)

    return pl.kernel(
        body,
        out_type=(
            jax.ShapeDtypeStruct((batch, EMBED), jnp.float32),
            jax.ShapeDtypeStruct((batch, EMBED), jnp.float32),
            jax.ShapeDtypeStruct((batch, EMBED), jnp.float32),
        ),
        mesh=plsc.VectorSubcoreMesh(core_axis_name="c", subcore_axis_name="s",
                                    num_cores=NC, num_subcores=NS),
        interpret=interpret,
    )


def _head_body(qp, pp, np_, Wq, bq, Wt, bt, left, right):
    qv = jnp.tanh(
        lax.dot_general(qp[...], Wq[...], (((1,), (1,)), ((), ())),
                        preferred_element_type=jnp.float32) + bq[...])
    pv = jnp.tanh(
        lax.dot_general(pp[...], Wt[...], (((1,), (1,)), ((), ())),
                        preferred_element_type=jnp.float32) + bt[...])
    nv = jnp.tanh(
        lax.dot_general(np_[...], Wt[...], (((1,), (1,)), ((), ())),
                        preferred_element_type=jnp.float32) + bt[...])
    nq = jnp.maximum(jnp.sqrt(jnp.sum(qv * qv, axis=1)), EPS)
    npv = jnp.maximum(jnp.sqrt(jnp.sum(pv * pv, axis=1)), EPS)
    nnv = jnp.maximum(jnp.sqrt(jnp.sum(nv * nv, axis=1)), EPS)
    left[...] = jnp.sum(qv * pv, axis=1) / (nq * npv)
    right[...] = jnp.sum(qv * nv, axis=1) / (nq * nnv)


@functools.lru_cache(maxsize=None)
def _build_head(batch, interpret=False):
    bt_tile = min(batch, 1024)
    grid = batch // bt_tile
    return pl.pallas_call(
        _head_body,
        grid=(grid,),
        in_specs=[
            pl.BlockSpec((bt_tile, EMBED), lambda i: (i, 0)),
            pl.BlockSpec((bt_tile, EMBED), lambda i: (i, 0)),
            pl.BlockSpec((bt_tile, EMBED), lambda i: (i, 0)),
            pl.BlockSpec((HIDDEN, EMBED), lambda i: (0, 0)),
            pl.BlockSpec((1, HIDDEN), lambda i: (0, 0)),
            pl.BlockSpec((HIDDEN, EMBED), lambda i: (0, 0)),
            pl.BlockSpec((1, HIDDEN), lambda i: (0, 0)),
        ],
        out_specs=[
            pl.BlockSpec((bt_tile,), lambda i: (i,)),
            pl.BlockSpec((bt_tile,), lambda i: (i,)),
        ],
        out_shape=[
            jax.ShapeDtypeStruct((batch,), jnp.float32),
            jax.ShapeDtypeStruct((batch,), jnp.float32),
        ],
        interpret=interpret,
    )


def _run(query, pos_title, neg_title, emb, Wq, bq, Wt, bt, interpret=False):
    batch = query.shape[0]
    ew = batch // NW
    pad = ((0, 0), (0, LTP - LT))
    qidx = query.reshape(NW, ew // GQ, GQ * LQ)
    pidx = jnp.pad(pos_title, pad).reshape(NW, ew // GT, GT * LTP)
    nidx = jnp.pad(neg_title, pad).reshape(NW, ew // GT, GT * LTP)
    q_pool, p_pool, n_pool = _build_sc_pool(batch, interpret)(
        emb, qidx, pidx, nidx)
    left, right = _build_head(batch, interpret)(
        q_pool, p_pool, n_pool,
        Wq, bq.reshape(1, HIDDEN), Wt, bt.reshape(1, HIDDEN))
    return (left, right)


@jax.jit
def kernel(query, pos_title, neg_title, emb, Wq, bq, Wt, bt):
    return _run(query, pos_title, neg_title, emb, Wq, bq, Wt, bt)


# speedup vs baseline: 10.0070x; 10.0070x over previous
import functools

import jax
import jax.numpy as jnp
from jax import lax
from jax.experimental import pallas as pl
from jax.experimental.pallas import tpu as pltpu
from jax.experimental.pallas import tpu_sc as plsc

EMBED = 256
HIDDEN = 128
LQ = 20
LT = 50
LTP = 52
EPS = 1e-08

NC = 2
NS = 16
NW = NC * NS

GQ = 8
GT = 4
NV = EMBED // 16


def _accumulate(rows_v, pool_v, base_row, elem, L):
    def body(r, accs):
        row = base_row + r
        return tuple(accs[v] + rows_v[row, pl.ds(v * 16, 16)]
                     for v in range(NV))
    init = tuple(rows_v[base_row, pl.ds(v * 16, 16)] for v in range(NV))
    accs = lax.fori_loop(1, L, body, init)
    for v in range(NV):
        pool_v[elem, pl.ds(v * 16, 16)] = accs[v]


def _pool_field(emb, idx_hbm, out_hbm, wid, ew, L, Lp, g, nchunks):
    R = g * Lp
    pieces = []
    ofs = 0
    while ofs < R:
        n = min(128, R - ofs)
        pieces.append((ofs, n))
        ofs += n

    def gather_chunk(idx_b, rows_v, gsem, b):
        for ofs, n in pieces:
            pltpu.async_copy(
                emb.at[idx_b.at[b, pl.ds(ofs, n)]],
                rows_v.at[b, pl.ds(ofs, n)], gsem.at[b])

    def wait_chunk(idx_b, rows_v, gsem, b):
        for ofs, n in pieces:
            pltpu.make_async_copy(
                emb.at[idx_b.at[b, pl.ds(ofs, n)]],
                rows_v.at[b, pl.ds(ofs, n)], gsem.at[b]).wait()

    def scoped(idx_b, rows_v, pool_v, gsem, osem, isem):
        for b in range(2):
            pltpu.sync_copy(idx_hbm.at[wid, b], idx_b.at[b])
            gather_chunk(idx_b, rows_v, gsem, b)

        def chunk_body(cc, carry):
            for b in range(2):
                c = cc * 2 + b
                wait_chunk(idx_b, rows_v, gsem, b)
                @pl.when(c + 2 < nchunks)
                def _():
                    pltpu.async_copy(
                        idx_hbm.at[wid, c + 2], idx_b.at[b], isem.at[b])
                @pl.when(c >= 2)
                def _():
                    pltpu.make_async_copy(
                        pool_v.at[b],
                        out_hbm.at[pl.ds(wid * ew + (c - 2) * g, g)],
                        osem.at[b]).wait()
                for e in range(g):
                    _accumulate(rows_v.at[b], pool_v.at[b], e * Lp, e, L)
                pltpu.async_copy(
                    pool_v.at[b],
                    out_hbm.at[pl.ds(wid * ew + c * g, g)],
                    osem.at[b])
                @pl.when(c + 2 < nchunks)
                def _():
                    pltpu.make_async_copy(
                        idx_hbm.at[wid, c + 2], idx_b.at[b], isem.at[b]).wait()
                    gather_chunk(idx_b, rows_v, gsem, b)
            return carry

        lax.fori_loop(0, nchunks // 2, chunk_body, 0)
        for b in range(2):
            c = nchunks - 2 + b
            pltpu.make_async_copy(
                pool_v.at[b],
                out_hbm.at[pl.ds(wid * ew + c * g, g)],
                osem.at[b]).wait()

    pl.run_scoped(
        scoped,
        pltpu.VMEM((2, R), jnp.int32),
        pltpu.VMEM((2, R, EMBED), jnp.float32),
        pltpu.VMEM((2, g, EMBED), jnp.float32),
        pltpu.SemaphoreType.DMA((2,)),
        pltpu.SemaphoreType.DMA((2,)),
        pltpu.SemaphoreType.DMA((2,)),
    )


@functools.lru_cache(maxsize=None)
def _build_sc_pool(batch, interpret=False):
    ew = batch // NW
    ncq = ew // GQ
    nct = ew // GT

    def body(emb, qidx, pidx, nidx, qout, pout, nout):
        wid = lax.axis_index("c") * NS + lax.axis_index("s")
        _pool_field(emb, qidx, qout, wid, ew, LQ, LQ, GQ, ncq)
        _pool_field(emb, pidx, pout, wid, ew, LT, LTP, GT, nct)
        _pool_field(emb, nidx, nout, wid, ew, LT, LTP, GT, nct)

    return pl.kernel(
        body,
        out_type=(
            jax.ShapeDtypeStruct((batch, EMBED), jnp.float32),
            jax.ShapeDtypeStruct((batch, EMBED), jnp.float32),
            jax.ShapeDtypeStruct((batch, EMBED), jnp.float32),
        ),
        mesh=plsc.VectorSubcoreMesh(core_axis_name="c", subcore_axis_name="s",
                                    num_cores=NC, num_subcores=NS),
        interpret=interpret,
    )


def _head_body(qp, pp, np_, Wq, bq, Wt, bt, left, right):
    qv = jnp.tanh(
        lax.dot_general(qp[...], Wq[...], (((1,), (1,)), ((), ())),
                        preferred_element_type=jnp.float32) + bq[...])
    pv = jnp.tanh(
        lax.dot_general(pp[...], Wt[...], (((1,), (1,)), ((), ())),
                        preferred_element_type=jnp.float32) + bt[...])
    nv = jnp.tanh(
        lax.dot_general(np_[...], Wt[...], (((1,), (1,)), ((), ())),
                        preferred_element_type=jnp.float32) + bt[...])
    nq = jnp.maximum(jnp.sqrt(jnp.sum(qv * qv, axis=1)), EPS)
    npv = jnp.maximum(jnp.sqrt(jnp.sum(pv * pv, axis=1)), EPS)
    nnv = jnp.maximum(jnp.sqrt(jnp.sum(nv * nv, axis=1)), EPS)
    left[...] = jnp.sum(qv * pv, axis=1) / (nq * npv)
    right[...] = jnp.sum(qv * nv, axis=1) / (nq * nnv)


@functools.lru_cache(maxsize=None)
def _build_head(batch, interpret=False):
    bt_tile = min(batch, 1024)
    grid = batch // bt_tile
    return pl.pallas_call(
        _head_body,
        grid=(grid,),
        in_specs=[
            pl.BlockSpec((bt_tile, EMBED), lambda i: (i, 0)),
            pl.BlockSpec((bt_tile, EMBED), lambda i: (i, 0)),
            pl.BlockSpec((bt_tile, EMBED), lambda i: (i, 0)),
            pl.BlockSpec((HIDDEN, EMBED), lambda i: (0, 0)),
            pl.BlockSpec((1, HIDDEN), lambda i: (0, 0)),
            pl.BlockSpec((HIDDEN, EMBED), lambda i: (0, 0)),
            pl.BlockSpec((1, HIDDEN), lambda i: (0, 0)),
        ],
        out_specs=[
            pl.BlockSpec((bt_tile,), lambda i: (i,)),
            pl.BlockSpec((bt_tile,), lambda i: (i,)),
        ],
        out_shape=[
            jax.ShapeDtypeStruct((batch,), jnp.float32),
            jax.ShapeDtypeStruct((batch,), jnp.float32),
        ],
        interpret=interpret,
    )


def _run_part(query, pos_title, neg_title, emb, Wq, bq, Wt, bt, interpret):
    batch = query.shape[0]
    ew = batch // NW
    pad = ((0, 0), (0, LTP - LT))
    qidx = query.reshape(NW, ew // GQ, GQ * LQ)
    pidx = jnp.pad(pos_title, pad, mode="wrap").reshape(NW, ew // GT, GT * LTP)
    nidx = jnp.pad(neg_title, pad, mode="wrap").reshape(NW, ew // GT, GT * LTP)
    q_pool, p_pool, n_pool = _build_sc_pool(batch, interpret)(
        emb, qidx, pidx, nidx)
    left, right = _build_head(batch, interpret)(
        q_pool, p_pool, n_pool,
        Wq, bq.reshape(1, HIDDEN), Wt, bt.reshape(1, HIDDEN))
    return (left, right)


def _run(query, pos_title, neg_title, emb, Wq, bq, Wt, bt, interpret=False):
    batch = query.shape[0]
    h = batch // 2
    if h % NW == 0 and (h // NW) % GQ == 0 and (h // NW) % GT == 0:
        outs = [
            _run_part(query[s], pos_title[s], neg_title[s],
                      emb, Wq, bq, Wt, bt, interpret)
            for s in (slice(0, h), slice(h, batch))
        ]
        return tuple(jnp.concatenate([o[i] for o in outs])
                     for i in range(2))
    return _run_part(query, pos_title, neg_title, emb, Wq, bq, Wt, bt,
                     interpret)


@jax.jit
def kernel(query, pos_title, neg_title, emb, Wq, bq, Wt, bt):
    return _run(query, pos_title, neg_title, emb, Wq, bq, Wt, bt)
